# Initial kernel scaffold; baseline (speedup 1.0000x reference)
import jax
import jax.numpy as jnp
from jax.experimental import pallas as pl


def kernel(input_tokens, input_seg, word_embeddings, segment_embeddings):
    raise NotImplementedError("write your pallas kernel here")


# speedup vs baseline: 5.8573x; 5.8573x over previous
import functools

import jax
import jax.numpy as jnp
from jax import lax
from jax.experimental import pallas as pl
from jax.experimental.pallas import tpu as pltpu
from jax.experimental.pallas import tpu_sc as plsc

NC = 2
NS = 16
LANES = 16


def _comb_table(segment_embeddings, seq_len):
    n_seg, d = segment_embeddings.shape

    def body(seg_ref, out_ref):
        s_idx = lax.broadcasted_iota(jnp.int32, (seq_len, d), 0)
        d_idx = lax.broadcasted_iota(jnp.int32, (seq_len, d), 1)
        i2 = ((d_idx // 2) * 2).astype(jnp.float32)
        div = jnp.exp(-jnp.log(10000.0) * i2 / d)
        ang = s_idx.astype(jnp.float32) * div
        pe = jnp.where(d_idx % 2 == 0, jnp.sin(ang), jnp.cos(ang))
        for t in range(n_seg):
            out_ref[pl.ds(t * seq_len, seq_len), :] = pe + seg_ref[t:t + 1, :]

    return pl.pallas_call(
        body,
        out_shape=jax.ShapeDtypeStruct((n_seg * seq_len, d), jnp.float32),
    )(segment_embeddings)


def _sc_embed(tok_flat, seg_flat, word_embeddings, comb, seq_len):
    n = tok_flat.shape[0]
    d = word_embeddings.shape[1]
    nw = NC * NS
    rows_per_w = n // nw
    ch = 400
    n_chunks = rows_per_w // ch
    assert rows_per_w % ch == 0 and ch % LANES == 0

    mesh = plsc.VectorSubcoreMesh(
        core_axis_name="c", subcore_axis_name="s",
        num_cores=NC, num_subcores=NS)

    @functools.partial(
        pl.kernel,
        out_type=jax.ShapeDtypeStruct((n, d), jnp.float32),
        mesh=mesh,
        scratch_types=[
            pltpu.VMEM((ch,), jnp.int32),
            pltpu.VMEM((ch,), jnp.int32),
            pltpu.VMEM((ch,), jnp.int32),
            pltpu.VMEM((ch, d), jnp.float32),
            pltpu.VMEM((ch, d), jnp.float32),
            pltpu.SemaphoreType.DMA,
            pltpu.SemaphoreType.DMA,
        ],
    )
    def k(tok_hbm, seg_hbm, wtab_hbm, comb_hbm, out_hbm,
          tokb, segb, cidxb, bufa, bufb, sema, semb):
        wid = lax.axis_index("s") * NC + lax.axis_index("c")
        base = wid * rows_per_w

        def chunk_body(c, carry):
            start = base + c * ch
            pltpu.sync_copy(tok_hbm.at[pl.ds(start, ch)], tokb)
            pltpu.sync_copy(seg_hbm.at[pl.ds(start, ch)], segb)

            def grp(g, cc):
                seg_v = segb[pl.ds(g * LANES, LANES)]
                i_v = lax.iota(jnp.int32, LANES) + g * LANES
                s_v = lax.rem(i_v, seq_len)
                cidxb[pl.ds(g * LANES, LANES)] = seg_v * seq_len + s_v
                return cc

            lax.fori_loop(0, ch // LANES, grp, 0)

            cpa = pltpu.async_copy(wtab_hbm.at[tokb], bufa, sema)
            cpb = pltpu.async_copy(comb_hbm.at[cidxb], bufb, semb)
            cpa.wait()
            cpb.wait()

            def addrow(r, cc):
                for j in range(d // LANES):
                    sl = pl.ds(j * LANES, LANES)
                    bufa[r, sl] = bufa[r, sl] + bufb[r, sl]
                return cc

            lax.fori_loop(0, ch, addrow, 0)
            pltpu.sync_copy(bufa, out_hbm.at[pl.ds(start, ch)])
            return carry

        lax.fori_loop(0, n_chunks, chunk_body, 0)

    return k(tok_flat, seg_flat, word_embeddings, comb)


def kernel(input_tokens, input_seg, word_embeddings, segment_embeddings):
    b, s = input_tokens.shape
    d = word_embeddings.shape[1]
    comb = _comb_table(segment_embeddings, s)
    tok_flat = input_tokens.reshape(-1).astype(jnp.int32)
    seg_flat = input_seg.reshape(-1).astype(jnp.int32)
    out = _sc_embed(tok_flat, seg_flat, word_embeddings, comb, s)
    return out.reshape(b, s, d)
